# baseline (speedup 1.0000x reference)
import functools
import numpy as np

import jax
import jax.numpy as jnp
from jax.experimental import pallas as pl
from jax.experimental.pallas import tpu as pltpu

_GH = 24
_GW = 24
_GD = 16
_REG_LAMBDA = 1e-7

_S_PAIRS = [(i, j) for i in range(4) for j in range(i, 4)]
_T_PAIRS = [(k, i) for k in range(3) for i in range(4)]
_NJ = len(_S_PAIRS) + len(_T_PAIRS)


def _spatial_weights(g, n):
    pos = (np.arange(n, dtype=np.float64) + 0.5) * (g - 1) / n
    i0 = np.clip(np.floor(pos).astype(np.int64), 0, g - 1)
    i1 = np.minimum(i0 + 1, g - 1)
    w = (pos - i0).astype(np.float32)
    m = np.zeros((g, n), dtype=np.float32)
    m[i0, np.arange(n)] += 1.0 - w
    m[i1, np.arange(n)] += w
    return m


def _split_bf16(x):
    hi = x.astype(jnp.bfloat16)
    lo = (x - hi.astype(jnp.float32)).astype(jnp.bfloat16)
    return hi, lo


def _fused_body(inp_ref, guide_ref, outp_ref, wgt_ref, wyh_ref, wyl_ref,
                wxth_ref, wxtl_ref, out_ref, ph_s, pl_s, g1_s,
                veh_s, vel_s, voh_s, vol_s):
    f32 = jnp.float32
    W = inp_ref.shape[3]
    a0 = inp_ref[0, 0]
    a1 = inp_ref[0, 1]
    a2 = inp_ref[0, 2]
    wmean = (wgt_ref[0, 0] + wgt_ref[0, 1] + wgt_ref[0, 2]) * (1.0 / 3.0)
    gz = guide_ref[0, 0] * (_GD - 1)
    z0 = jnp.clip(jnp.floor(gz).astype(jnp.int32), 0, _GD - 1)
    z1 = jnp.minimum(z0 + 1, _GD - 1)
    wz = gz - z0.astype(f32)
    a = (a0, a1, a2, None)
    o = (outp_ref[0, 0], outp_ref[0, 1], outp_ref[0, 2])
    wyh, wyl = wyh_ref[...], wyl_ref[...]
    wxth, wxtl = wxth_ref[...], wxtl_ref[...]

    wz0 = 1.0 - wz
    pe = (z0 & 1) == 0
    zevb = jnp.where(pe, z0, z1).astype(jnp.bfloat16)
    zodb = jnp.where(pe, z1, z0).astype(jnp.bfloat16)
    specs = [(a[i1], a[i2]) for (i1, i2) in _S_PAIRS] + \
            [(o[k], a[i1]) for (k, i1) in _T_PAIRS]
    for j, (f1, f2) in enumerate(specs):
        sl = slice(j * W, (j + 1) * W)
        if f1 is None and f2 is None:
            src = wmean
        elif f2 is None:
            src = f1 * wmean
        elif f1 is None:
            src = f2 * wmean
        else:
            src = (f1 * f2) * wmean
        v0h, v0l = _split_bf16(src * wz0)
        v1h, v1l = _split_bf16(src * wz)
        veh_s[:, sl] = jnp.where(pe, v0h, v1h)
        vel_s[:, sl] = jnp.where(pe, v0l, v1l)
        voh_s[:, sl] = jnp.where(pe, v1h, v0h)
        vol_s[:, sl] = jnp.where(pe, v1l, v0l)
    zero_b = jnp.zeros_like(zevb)

    groups = [(0, 8), (8, 8), (16, _NJ - 16)]

    def pair_step(k, _):
        for vh_s, vl_s, ztb, zeta in ((veh_s, vel_s, zevb, 2 * k),
                                      (voh_s, vol_s, zodb, 2 * k + 1)):
            m = ztb == zeta.astype(jnp.bfloat16)
            for g0, gw in groups:
                for jj in range(gw):
                    sl = slice((g0 + jj) * W, (g0 + jj + 1) * W)
                    dl = slice(jj * W, (jj + 1) * W)
                    ph_s[:, dl] = jnp.where(m, vh_s[:, sl], zero_b)
                    pl_s[:, dl] = jnp.where(m, vl_s[:, sl], zero_b)
                phv = ph_s[:, :gw * W]
                plv = pl_s[:, :gw * W]
                g1 = (jnp.dot(wyh, phv, preferred_element_type=f32)
                      + jnp.dot(wyh, plv, preferred_element_type=f32)
                      + jnp.dot(wyl, phv, preferred_element_type=f32))
                g1_s[pl.ds(zeta * _GH, _GH), g0 * W:(g0 + gw) * W] = g1
        return 0

    jax.lax.fori_loop(0, _GD // 2, pair_step, 0, unroll=False)

    ncell = _GD * _GH * _GW
    g_arrs = []
    for j in range(_NJ):
        q = g1_s[:, j * W:(j + 1) * W]
        qh, ql = _split_bf16(q)
        r = (jnp.dot(qh, wxth, preferred_element_type=f32)
             + jnp.dot(qh, wxtl, preferred_element_type=f32)
             + jnp.dot(ql, wxth, preferred_element_type=f32))
        g_arrs.append(r.reshape(_GD, _GH, _GW))

    _solve_from(g_arrs, out_ref)


def _inv4_sym(m):
    s0 = m[0, 0] * m[1, 1] - m[1, 0] * m[0, 1]
    s1 = m[0, 0] * m[1, 2] - m[1, 0] * m[0, 2]
    s2 = m[0, 0] * m[1, 3] - m[1, 0] * m[0, 3]
    s3 = m[0, 1] * m[1, 2] - m[1, 1] * m[0, 2]
    s4 = m[0, 1] * m[1, 3] - m[1, 1] * m[0, 3]
    s5 = m[0, 2] * m[1, 3] - m[1, 2] * m[0, 3]
    c5 = m[2, 2] * m[3, 3] - m[3, 2] * m[2, 3]
    c4 = m[2, 1] * m[3, 3] - m[3, 1] * m[2, 3]
    c3 = m[2, 1] * m[3, 2] - m[3, 1] * m[2, 2]
    c2 = m[2, 0] * m[3, 3] - m[3, 0] * m[2, 3]
    c1 = m[2, 0] * m[3, 2] - m[3, 0] * m[2, 2]
    c0 = m[2, 0] * m[3, 1] - m[3, 0] * m[2, 1]
    det = s0 * c5 - s1 * c4 + s2 * c3 + s3 * c2 - s4 * c1 + s5 * c0
    rdet = 1.0 / det
    inv = {}
    inv[0, 0] = (m[1, 1] * c5 - m[1, 2] * c4 + m[1, 3] * c3) * rdet
    inv[0, 1] = (-m[0, 1] * c5 + m[0, 2] * c4 - m[0, 3] * c3) * rdet
    inv[0, 2] = (m[3, 1] * s5 - m[3, 2] * s4 + m[3, 3] * s3) * rdet
    inv[0, 3] = (-m[2, 1] * s5 + m[2, 2] * s4 - m[2, 3] * s3) * rdet
    inv[1, 0] = (-m[1, 0] * c5 + m[1, 2] * c2 - m[1, 3] * c1) * rdet
    inv[1, 1] = (m[0, 0] * c5 - m[0, 2] * c2 + m[0, 3] * c1) * rdet
    inv[1, 2] = (-m[3, 0] * s5 + m[3, 2] * s2 - m[3, 3] * s1) * rdet
    inv[1, 3] = (m[2, 0] * s5 - m[2, 2] * s2 + m[2, 3] * s1) * rdet
    inv[2, 0] = (m[1, 0] * c4 - m[1, 1] * c2 + m[1, 3] * c0) * rdet
    inv[2, 1] = (-m[0, 0] * c4 + m[0, 1] * c2 - m[0, 3] * c0) * rdet
    inv[2, 2] = (m[3, 0] * s4 - m[3, 1] * s2 + m[3, 3] * s0) * rdet
    inv[2, 3] = (-m[2, 0] * s4 + m[2, 1] * s2 - m[2, 3] * s0) * rdet
    inv[3, 0] = (-m[1, 0] * c3 + m[1, 1] * c1 - m[1, 2] * c0) * rdet
    inv[3, 1] = (m[0, 0] * c3 - m[0, 1] * c1 + m[0, 2] * c0) * rdet
    inv[3, 2] = (-m[3, 0] * s3 + m[3, 1] * s1 - m[3, 2] * s0) * rdet
    inv[3, 3] = (m[2, 0] * s3 - m[2, 1] * s1 + m[2, 2] * s0) * rdet
    return inv


def _solve_from(g_arrs, out_ref):
    S = {}
    for j, (i1, i2) in enumerate(_S_PAIRS):
        S[i1, i2] = g_arrs[j]
        S[i2, i1] = S[i1, i2]
    T = {}
    for j, (k, i1) in enumerate(_T_PAIRS):
        T[k, i1] = g_arrs[len(_S_PAIRS) + j]

    counts = S[3, 3]
    wl = _REG_LAMBDA * (counts + 1.0)

    gcs = jnp.sum(counts)
    wlg = _REG_LAMBDA * (gcs + 1.0)
    gain_g = [jnp.sum(T[k, 3]) / (jnp.sum(S[k, 3]) + wlg) for k in range(3)]
    zero_mask = counts == 0.0
    mixed = [jnp.where(zero_mask, gain_g[k], T[k, 3] / (S[k, 3] + wl))
             for k in range(3)]

    Sr = {}
    for i in range(4):
        for j in range(4):
            Sr[i, j] = S[i, j] + wl if i == j else S[i, j]
    Tr = {}
    for k in range(3):
        for i in range(4):
            Tr[k, i] = T[k, i] + wl * mixed[k] if i == k else T[k, i]

    amax = Sr[0, 0]
    for i in range(4):
        for j in range(4):
            amax = jnp.maximum(amax, jnp.abs(Sr[i, j]))
    scale = 1.0 / amax
    Sn = {k: v * scale for k, v in Sr.items()}
    inv = _inv4_sym(Sn)
    for k in range(3):
        for i in range(4):
            acc = Tr[k, 0] * inv[0, i]
            for q in range(1, 4):
                acc = acc + Tr[k, q] * inv[q, i]
            out_ref[k, i] = acc * scale


@jax.jit
def kernel(input_image, guide_image, output_image, weight_image):
    B, C, H, W = input_image.shape
    dtype = input_image.dtype
    wy = jnp.asarray(_spatial_weights(_GH, H))
    wxt = jnp.asarray(_spatial_weights(_GW, W).T)
    wyh, wyl = _split_bf16(wy)
    wxth, wxtl = _split_bf16(wxt)

    gamma = pl.pallas_call(
        _fused_body,
        out_shape=jax.ShapeDtypeStruct((3, 4, _GD, _GH, _GW), dtype),
        compiler_params=pltpu.CompilerParams(
            vmem_limit_bytes=63 * 1024 * 1024),
        scratch_shapes=[
            pltpu.VMEM((H, 8 * W), jnp.bfloat16),
            pltpu.VMEM((H, 8 * W), jnp.bfloat16),
            pltpu.VMEM((_GD * _GH, _NJ * W), dtype),
            pltpu.VMEM((H, _NJ * W), jnp.bfloat16),
            pltpu.VMEM((H, _NJ * W), jnp.bfloat16),
            pltpu.VMEM((H, _NJ * W), jnp.bfloat16),
            pltpu.VMEM((H, _NJ * W), jnp.bfloat16),
        ],
    )(input_image, guide_image, output_image, weight_image,
      wyh, wyl, wxth, wxtl)

    return jnp.transpose(gamma, (3, 4, 2, 0, 1))[None]


# speedup vs baseline: 1.0002x; 1.0002x over previous
import functools
import numpy as np

import jax
import jax.numpy as jnp
from jax.experimental import pallas as pl
from jax.experimental.pallas import tpu as pltpu

_GH = 24
_GW = 24
_GD = 16
_REG_LAMBDA = 1e-7

_S_PAIRS = [(i, j) for i in range(4) for j in range(i, 4)]
_T_PAIRS = [(k, i) for k in range(3) for i in range(4)]
_NJ = len(_S_PAIRS) + len(_T_PAIRS)


def _spatial_weights(g, n):
    pos = (np.arange(n, dtype=np.float64) + 0.5) * (g - 1) / n
    i0 = np.clip(np.floor(pos).astype(np.int64), 0, g - 1)
    i1 = np.minimum(i0 + 1, g - 1)
    w = (pos - i0).astype(np.float32)
    m = np.zeros((g, n), dtype=np.float32)
    m[i0, np.arange(n)] += 1.0 - w
    m[i1, np.arange(n)] += w
    return m


def _split_bf16(x):
    hi = x.astype(jnp.bfloat16)
    lo = (x - hi.astype(jnp.float32)).astype(jnp.bfloat16)
    return hi, lo


def _fused_body(inp_ref, guide_ref, outp_ref, wgt_ref, wyh_ref, wyl_ref,
                wxth_ref, wxtl_ref, out_ref, ph_s, pl_s, g1_s):
    f32 = jnp.float32
    W = inp_ref.shape[3]
    a0 = inp_ref[0, 0]
    a1 = inp_ref[0, 1]
    a2 = inp_ref[0, 2]
    wmean = (wgt_ref[0, 0] + wgt_ref[0, 1] + wgt_ref[0, 2]) * (1.0 / 3.0)
    gz = guide_ref[0, 0] * (_GD - 1)
    z0 = jnp.clip(jnp.floor(gz).astype(jnp.int32), 0, _GD - 1)
    z1 = jnp.minimum(z0 + 1, _GD - 1)
    wz = gz - z0.astype(f32)
    a = (a0, a1, a2, None)
    o = (outp_ref[0, 0], outp_ref[0, 1], outp_ref[0, 2])
    wyh, wyl = wyh_ref[...], wyl_ref[...]
    wxth, wxtl = wxth_ref[...], wxtl_ref[...]

    specs = [(a[i1], a[i2]) for (i1, i2) in _S_PAIRS] + \
            [(o[k], a[i1]) for (k, i1) in _T_PAIRS]
    v = []
    for f1, f2 in specs:
        if f1 is None and f2 is None:
            v.append(wmean)
        elif f2 is None:
            v.append(f1 * wmean)
        elif f1 is None:
            v.append(f2 * wmean)
        else:
            v.append((f1 * f2) * wmean)

    def zeta_step(zeta, _):
        uz = jnp.where(z0 == zeta, 1.0 - wz, 0.0) + jnp.where(z1 == zeta, wz, 0.0)
        for j in range(_NJ):
            p = v[j] * uz
            ph, plo = _split_bf16(p)
            ph_s[:, j * W:(j + 1) * W] = ph
            pl_s[:, j * W:(j + 1) * W] = plo
        phv = ph_s[...]
        plv = pl_s[...]
        g1 = (jnp.dot(wyh, phv, preferred_element_type=f32)
              + jnp.dot(wyh, plv, preferred_element_type=f32)
              + jnp.dot(wyl, phv, preferred_element_type=f32))
        g1_s[pl.ds(zeta * _GH, _GH), :] = g1
        return 0

    jax.lax.fori_loop(0, _GD, zeta_step, 0, unroll=False)

    ncell = _GD * _GH * _GW
    g_arrs = []
    for j in range(_NJ):
        q = g1_s[:, j * W:(j + 1) * W]
        qh, ql = _split_bf16(q)
        r = (jnp.dot(qh, wxth, preferred_element_type=f32)
             + jnp.dot(qh, wxtl, preferred_element_type=f32)
             + jnp.dot(ql, wxth, preferred_element_type=f32))
        g_arrs.append(r.reshape(_GD, _GH, _GW))

    _solve_from(g_arrs, out_ref)


def _inv4_sym(m):
    s0 = m[0, 0] * m[1, 1] - m[1, 0] * m[0, 1]
    s1 = m[0, 0] * m[1, 2] - m[1, 0] * m[0, 2]
    s2 = m[0, 0] * m[1, 3] - m[1, 0] * m[0, 3]
    s3 = m[0, 1] * m[1, 2] - m[1, 1] * m[0, 2]
    s4 = m[0, 1] * m[1, 3] - m[1, 1] * m[0, 3]
    s5 = m[0, 2] * m[1, 3] - m[1, 2] * m[0, 3]
    c5 = m[2, 2] * m[3, 3] - m[3, 2] * m[2, 3]
    c4 = m[2, 1] * m[3, 3] - m[3, 1] * m[2, 3]
    c3 = m[2, 1] * m[3, 2] - m[3, 1] * m[2, 2]
    c2 = m[2, 0] * m[3, 3] - m[3, 0] * m[2, 3]
    c1 = m[2, 0] * m[3, 2] - m[3, 0] * m[2, 2]
    c0 = m[2, 0] * m[3, 1] - m[3, 0] * m[2, 1]
    det = s0 * c5 - s1 * c4 + s2 * c3 + s3 * c2 - s4 * c1 + s5 * c0
    rdet = 1.0 / det
    inv = {}
    inv[0, 0] = (m[1, 1] * c5 - m[1, 2] * c4 + m[1, 3] * c3) * rdet
    inv[0, 1] = (-m[0, 1] * c5 + m[0, 2] * c4 - m[0, 3] * c3) * rdet
    inv[0, 2] = (m[3, 1] * s5 - m[3, 2] * s4 + m[3, 3] * s3) * rdet
    inv[0, 3] = (-m[2, 1] * s5 + m[2, 2] * s4 - m[2, 3] * s3) * rdet
    inv[1, 0] = (-m[1, 0] * c5 + m[1, 2] * c2 - m[1, 3] * c1) * rdet
    inv[1, 1] = (m[0, 0] * c5 - m[0, 2] * c2 + m[0, 3] * c1) * rdet
    inv[1, 2] = (-m[3, 0] * s5 + m[3, 2] * s2 - m[3, 3] * s1) * rdet
    inv[1, 3] = (m[2, 0] * s5 - m[2, 2] * s2 + m[2, 3] * s1) * rdet
    inv[2, 0] = (m[1, 0] * c4 - m[1, 1] * c2 + m[1, 3] * c0) * rdet
    inv[2, 1] = (-m[0, 0] * c4 + m[0, 1] * c2 - m[0, 3] * c0) * rdet
    inv[2, 2] = (m[3, 0] * s4 - m[3, 1] * s2 + m[3, 3] * s0) * rdet
    inv[2, 3] = (-m[2, 0] * s4 + m[2, 1] * s2 - m[2, 3] * s0) * rdet
    inv[3, 0] = (-m[1, 0] * c3 + m[1, 1] * c1 - m[1, 2] * c0) * rdet
    inv[3, 1] = (m[0, 0] * c3 - m[0, 1] * c1 + m[0, 2] * c0) * rdet
    inv[3, 2] = (-m[3, 0] * s3 + m[3, 1] * s1 - m[3, 2] * s0) * rdet
    inv[3, 3] = (m[2, 0] * s3 - m[2, 1] * s1 + m[2, 2] * s0) * rdet
    return inv


def _solve_from(g_arrs, out_ref):
    S = {}
    for j, (i1, i2) in enumerate(_S_PAIRS):
        S[i1, i2] = g_arrs[j]
        S[i2, i1] = S[i1, i2]
    T = {}
    for j, (k, i1) in enumerate(_T_PAIRS):
        T[k, i1] = g_arrs[len(_S_PAIRS) + j]

    counts = S[3, 3]
    wl = _REG_LAMBDA * (counts + 1.0)

    gcs = jnp.sum(counts)
    wlg = _REG_LAMBDA * (gcs + 1.0)
    gain_g = [jnp.sum(T[k, 3]) / (jnp.sum(S[k, 3]) + wlg) for k in range(3)]
    zero_mask = counts == 0.0
    mixed = [jnp.where(zero_mask, gain_g[k], T[k, 3] / (S[k, 3] + wl))
             for k in range(3)]

    Sr = {}
    for i in range(4):
        for j in range(4):
            Sr[i, j] = S[i, j] + wl if i == j else S[i, j]
    Tr = {}
    for k in range(3):
        for i in range(4):
            Tr[k, i] = T[k, i] + wl * mixed[k] if i == k else T[k, i]

    amax = Sr[0, 0]
    for i in range(4):
        for j in range(4):
            amax = jnp.maximum(amax, jnp.abs(Sr[i, j]))
    scale = 1.0 / amax
    Sn = {k: v * scale for k, v in Sr.items()}
    inv = _inv4_sym(Sn)
    for k in range(3):
        for i in range(4):
            acc = Tr[k, 0] * inv[0, i]
            for q in range(1, 4):
                acc = acc + Tr[k, q] * inv[q, i]
            out_ref[k, i] = acc * scale


@jax.jit
def kernel(input_image, guide_image, output_image, weight_image):
    B, C, H, W = input_image.shape
    dtype = input_image.dtype
    wy = jnp.asarray(_spatial_weights(_GH, H))
    wxt = jnp.asarray(_spatial_weights(_GW, W).T)
    wyh, wyl = _split_bf16(wy)
    wxth, wxtl = _split_bf16(wxt)

    gamma = pl.pallas_call(
        _fused_body,
        out_shape=jax.ShapeDtypeStruct((3, 4, _GD, _GH, _GW), dtype),
        compiler_params=pltpu.CompilerParams(
            vmem_limit_bytes=63 * 1024 * 1024),
        scratch_shapes=[
            pltpu.VMEM((H, _NJ * W), jnp.bfloat16),
            pltpu.VMEM((H, _NJ * W), jnp.bfloat16),
            pltpu.VMEM((_GD * _GH, _NJ * W), dtype),
        ],
    )(input_image, guide_image, output_image, weight_image,
      wyh, wyl, wxth, wxtl)

    return jnp.transpose(gamma, (3, 4, 2, 0, 1))[None]
